# Initial kernel scaffold; baseline (speedup 1.0000x reference)
import jax
import jax.numpy as jnp
from jax.experimental import pallas as pl


def kernel(gene_ids, expression_tokens, condition_tokens, library_size, gene_table, expr_table, cond_table, lib_table, pos_table, W_mix, b_mix, ln_gamma, ln_beta):
    raise NotImplementedError("write your pallas kernel here")


# speedup vs baseline: 4.0880x; 4.0880x over previous
import functools

import jax
import jax.numpy as jnp
from jax import lax
from jax.experimental import pallas as pl
from jax.experimental.pallas import tpu as pltpu
from jax.experimental.pallas import tpu_sc as plsc

B = 1024
L = 512
D = 64
GENE_V = 100000
EXPR_V = 1000

NC = 2
NS = 16
NW = NC * NS
TOK = B * L
TPW = TOK // NW
CH = 256
NCHUNK = TPW // CH
SUB = 128
CPW = B // NW


def _premix_gene_body(g_ref, w_ref, o_ref):
    w1 = w_ref[...][:, :D]
    h = lax.dot_general(g_ref[...], w1, (((1,), (1,)), ((), ())),
                        preferred_element_type=jnp.float32)
    o_ref[...] = jnp.concatenate([h, jnp.zeros_like(h)], axis=1)


def _premix_small_body(e_ref, w_ref, b_ref, c_ref, l_ref, eo_ref, co_ref,
                       lo_ref):
    w2 = w_ref[...][:, D:]
    h = lax.dot_general(e_ref[...], w2, (((1,), (1,)), ((), ())),
                        preferred_element_type=jnp.float32) + b_ref[...]
    z = jnp.zeros_like(h)
    eo_ref[...] = jnp.concatenate([z, h], axis=1)
    co_ref[...] = jnp.concatenate([c_ref[...], z], axis=1)
    lo_ref[...] = jnp.concatenate([z, l_ref[...]], axis=1)


def _finish_body(s_ref, pos_ref, extra_ref, gamma_ref, beta_ref, o_ref):
    x = jnp.maximum(s_ref[...], 0.0)
    mean = jnp.mean(x, axis=-1, keepdims=True)
    xc = x - mean
    var = jnp.mean(xc * xc, axis=-1, keepdims=True)
    y = xc * lax.rsqrt(var + 1e-5) * gamma_ref[...] + beta_ref[...]
    y = y + pos_ref[...]
    lmask = (lax.broadcasted_iota(jnp.int32, (1, L, 1), 1) == 0)
    y = y + jnp.where(lmask, 1.0, 0.0) * extra_ref[...][:, None, :]
    o_ref[...] = y


def _sc_gather_add(gm_hbm, em_hbm, gi_hbm, ei_hbm, ct_hbm, lt_hbm, ci_hbm,
                   li_hbm, s_hbm, extra_hbm,
                   gidx_v, eidx_v, rows_g, rows_e, rows_o, cidx_v, lidx_v,
                   crow, lrow, orow, sem):
    wid = lax.axis_index("s") * NC + lax.axis_index("c")

    cb = wid * CPW
    pltpu.sync_copy(ci_hbm.at[pl.ds(cb, CPW)], cidx_v)
    pltpu.sync_copy(li_hbm.at[pl.ds(cb, CPW)], lidx_v)
    pltpu.async_copy(ct_hbm.at[cidx_v], crow, sem).wait()
    pltpu.async_copy(lt_hbm.at[lidx_v], lrow, sem).wait()

    @pl.loop(0, CPW)
    def _(r):
        for i in range(D // 16):
            sl = pl.ds(i * 16, 16)
            sh = pl.ds(D + i * 16, 16)
            orow[r, sl] = crow[r, sl] + lrow[r, sh]

    pltpu.sync_copy(orow, extra_hbm.at[pl.ds(cb, CPW)])

    base0 = wid * TPW

    @pl.loop(0, NCHUNK)
    def _(ch):
        base = base0 + ch * CH
        pltpu.sync_copy(gi_hbm.at[pl.ds(base, CH)], gidx_v)
        pltpu.sync_copy(ei_hbm.at[pl.ds(base, CH)], eidx_v)
        copies = []
        for j in range(CH // SUB):
            sl = pl.ds(j * SUB, SUB)
            copies.append(pltpu.async_copy(
                gm_hbm.at[gidx_v.at[sl]], rows_g.at[sl], sem))
            copies.append(pltpu.async_copy(
                em_hbm.at[eidx_v.at[sl]], rows_e.at[sl], sem))
        for c in copies:
            c.wait()

        @pl.loop(0, CH)
        def _(r):
            for i in range(D // 16):
                sl = pl.ds(i * 16, 16)
                sh = pl.ds(D + i * 16, 16)
                rows_o[r, sl] = rows_g[r, sl] + rows_e[r, sh]

        pltpu.sync_copy(rows_o, s_hbm.at[pl.ds(base, CH)])


@functools.cache
def _sc_gather_call():
    return functools.partial(
        pl.kernel,
        out_type=(jax.ShapeDtypeStruct((TOK, D), jnp.float32),
                  jax.ShapeDtypeStruct((B, D), jnp.float32)),
        mesh=plsc.VectorSubcoreMesh(core_axis_name="c", subcore_axis_name="s"),
        scratch_types=[
            pltpu.VMEM((CH,), jnp.int32),
            pltpu.VMEM((CH,), jnp.int32),
            pltpu.VMEM((CH, 2 * D), jnp.float32),
            pltpu.VMEM((CH, 2 * D), jnp.float32),
            pltpu.VMEM((CH, D), jnp.float32),
            pltpu.VMEM((CPW,), jnp.int32),
            pltpu.VMEM((CPW,), jnp.int32),
            pltpu.VMEM((CPW, 2 * D), jnp.float32),
            pltpu.VMEM((CPW, 2 * D), jnp.float32),
            pltpu.VMEM((CPW, D), jnp.float32),
            pltpu.SemaphoreType.DMA,
        ],
    )(_sc_gather_add)


GBLK = 4000
RB = 8


def kernel(gene_ids, expression_tokens, condition_tokens, library_size,
           gene_table, expr_table, cond_table, lib_table, pos_table,
           W_mix, b_mix, ln_gamma, ln_beta):
    gi = jnp.asarray(gene_ids, jnp.int32).reshape(TOK)
    ei = jnp.asarray(expression_tokens, jnp.int32).reshape(TOK)
    ci = jnp.asarray(condition_tokens, jnp.int32)
    li = jnp.asarray(library_size, jnp.int32)

    gene_mixed = pl.pallas_call(
        _premix_gene_body,
        grid=(GENE_V // GBLK,),
        in_specs=[pl.BlockSpec((GBLK, D), lambda i: (i, 0)),
                  pl.BlockSpec((D, 2 * D), lambda i: (0, 0))],
        out_specs=pl.BlockSpec((GBLK, 2 * D), lambda i: (i, 0)),
        out_shape=jax.ShapeDtypeStruct((GENE_V, 2 * D), jnp.float32),
    )(gene_table, W_mix)

    expr_mixed, cond_wide, lib_wide = pl.pallas_call(
        _premix_small_body,
        grid=(1,),
        in_specs=[pl.BlockSpec((EXPR_V, D), lambda i: (0, 0)),
                  pl.BlockSpec((D, 2 * D), lambda i: (0, 0)),
                  pl.BlockSpec((1, D), lambda i: (0, 0)),
                  pl.BlockSpec((EXPR_V, D), lambda i: (0, 0)),
                  pl.BlockSpec((EXPR_V, D), lambda i: (0, 0))],
        out_specs=[pl.BlockSpec((EXPR_V, 2 * D), lambda i: (0, 0)),
                   pl.BlockSpec((EXPR_V, 2 * D), lambda i: (0, 0)),
                   pl.BlockSpec((EXPR_V, 2 * D), lambda i: (0, 0))],
        out_shape=[jax.ShapeDtypeStruct((EXPR_V, 2 * D), jnp.float32),
                   jax.ShapeDtypeStruct((EXPR_V, 2 * D), jnp.float32),
                   jax.ShapeDtypeStruct((EXPR_V, 2 * D), jnp.float32)],
    )(expr_table, W_mix, b_mix.reshape(1, D), cond_table, lib_table)

    s, extra = _sc_gather_call()(gene_mixed, expr_mixed, gi, ei,
                                 cond_wide, lib_wide, ci, li)

    out = pl.pallas_call(
        _finish_body,
        grid=(B // RB,),
        in_specs=[pl.BlockSpec((RB, L, D), lambda i: (i, 0, 0)),
                  pl.BlockSpec((L, D), lambda i: (0, 0)),
                  pl.BlockSpec((RB, D), lambda i: (i, 0)),
                  pl.BlockSpec((1, D), lambda i: (0, 0)),
                  pl.BlockSpec((1, D), lambda i: (0, 0))],
        out_specs=pl.BlockSpec((RB, L, D), lambda i: (i, 0, 0)),
        out_shape=jax.ShapeDtypeStruct((B, L, D), jnp.float32),
    )(s.reshape(B, L, D), pos_table, extra,
      ln_gamma.reshape(1, D), ln_beta.reshape(1, D))

    return out
